# baseline (speedup 1.0000x reference)
import functools

import jax
import jax.numpy as jnp
from jax import lax
from jax.experimental import pallas as pl
from jax.experimental.pallas import tpu as pltpu
from jax.experimental.pallas import tpu_sc as plsc

N = 10000
E = 320000
D = 128

NC = 2
NS = 16
NW = NC * NS
EPW = E // NW
CHUNK = 400
NCHUNK = EPW // CHUNK


def _bdot(a, b):
    return jnp.dot(a.astype(jnp.bfloat16), b.astype(jnp.bfloat16),
                   preferred_element_type=jnp.float32)


def _node_proj_body(x_ref, wa_ref, wb_ref, pa_ref, pb_ref):
    x = x_ref[...]
    pa_ref[...] = _bdot(x, wa_ref[...])
    pb_ref[...] = _bdot(x, wb_ref[...])


def _node_proj(node_x, wa, wb):
    bn = 1000
    grid = (N // bn,)
    return pl.pallas_call(
        _node_proj_body,
        grid=grid,
        in_specs=[
            pl.BlockSpec((bn, D), lambda i: (i, 0)),
            pl.BlockSpec((D, D), lambda i: (0, 0)),
            pl.BlockSpec((D, D), lambda i: (0, 0)),
        ],
        out_specs=[
            pl.BlockSpec((bn, D), lambda i: (i, 0)),
            pl.BlockSpec((bn, D), lambda i: (i, 0)),
        ],
        out_shape=[
            jax.ShapeDtypeStruct((N, D), jnp.float32),
            jax.ShapeDtypeStruct((N, D), jnp.float32),
        ],
    )(node_x, wa, wb)


def _ln_affine(h, g, b):
    mu = jnp.mean(h, axis=-1, keepdims=True)
    hc = h - mu
    var = jnp.mean(hc * hc, axis=-1, keepdims=True)
    return hc * lax.rsqrt(var + 1e-5) * g + b


def _edge_mlp_body(ga_ref, gb_ref, e_ref, wc_ref, w1_ref, w2_ref,
                   b0_ref, b1_ref, b2_ref, g_ref, beta_ref, out_ref):
    e = e_ref[...]
    h0 = ga_ref[...] + gb_ref[...] + _bdot(e, wc_ref[...]) + b0_ref[...]
    h0 = jnp.maximum(h0, 0.0)
    h1 = jnp.maximum(_bdot(h0, w1_ref[...]) + b1_ref[...], 0.0)
    h2 = _bdot(h1, w2_ref[...]) + b2_ref[...]
    out_ref[...] = e + _ln_affine(h2, g_ref[...], beta_ref[...])


def _edge_mlp(ga, gb, edge_attr, wc, w1, w2, b0, b1, b2, g, beta):
    be = 512
    grid = (E // be,)
    row = lambda i: (i, 0)
    fix = lambda i: (0, 0)
    return pl.pallas_call(
        _edge_mlp_body,
        grid=grid,
        in_specs=[
            pl.BlockSpec((be, D), row),
            pl.BlockSpec((be, D), row),
            pl.BlockSpec((be, D), row),
            pl.BlockSpec((D, D), fix),
            pl.BlockSpec((D, D), fix),
            pl.BlockSpec((D, D), fix),
            pl.BlockSpec((1, D), fix),
            pl.BlockSpec((1, D), fix),
            pl.BlockSpec((1, D), fix),
            pl.BlockSpec((1, D), fix),
            pl.BlockSpec((1, D), fix),
        ],
        out_specs=pl.BlockSpec((be, D), row),
        out_shape=jax.ShapeDtypeStruct((E, D), jnp.float32),
    )(ga, gb, edge_attr, wc, w1, w2, b0, b1, b2, g, beta)


def _node_update_body(x_ref, p0_ref, p1_ref, wa_ref, wb_ref, w1_ref, w2_ref,
                      b0_ref, b1_ref, b2_ref, g_ref, beta_ref, out_ref):
    x = x_ref[...]
    agg = p0_ref[...] + p1_ref[...]
    h0 = _bdot(x, wa_ref[...]) + _bdot(agg, wb_ref[...]) + b0_ref[...]
    h0 = jnp.maximum(h0, 0.0)
    h1 = jnp.maximum(_bdot(h0, w1_ref[...]) + b1_ref[...], 0.0)
    h2 = _bdot(h1, w2_ref[...]) + b2_ref[...]
    out_ref[...] = x + _ln_affine(h2, g_ref[...], beta_ref[...])


def _node_update(node_x, p0, p1, wa, wb, w1, w2, b0, b1, b2, g, beta):
    bn = 1000
    grid = (N // bn,)
    row = lambda i: (i, 0)
    fix = lambda i: (0, 0)
    return pl.pallas_call(
        _node_update_body,
        grid=grid,
        in_specs=[
            pl.BlockSpec((bn, D), row),
            pl.BlockSpec((bn, D), row),
            pl.BlockSpec((bn, D), row),
            pl.BlockSpec((D, D), fix),
            pl.BlockSpec((D, D), fix),
            pl.BlockSpec((D, D), fix),
            pl.BlockSpec((D, D), fix),
            pl.BlockSpec((1, D), fix),
            pl.BlockSpec((1, D), fix),
            pl.BlockSpec((1, D), fix),
            pl.BlockSpec((1, D), fix),
            pl.BlockSpec((1, D), fix),
        ],
        out_specs=pl.BlockSpec((bn, D), row),
        out_shape=jax.ShapeDtypeStruct((N, D), jnp.float32),
    )(node_x, p0, p1, wa, wb, w1, w2, b0, b1, b2, g, beta)


@functools.cache
def _sc_gather_kernel():
    mesh = plsc.VectorSubcoreMesh(core_axis_name="c", subcore_axis_name="s",
                                  num_cores=NC, num_subcores=NS)

    @functools.partial(
        pl.kernel,
        out_type=[
            jax.ShapeDtypeStruct((E, D), jnp.float32),
            jax.ShapeDtypeStruct((E, D), jnp.float32),
        ],
        mesh=mesh,
        scratch_types=[
            pltpu.VMEM((CHUNK,), jnp.int32),
            pltpu.VMEM((CHUNK, D), jnp.float32),
            pltpu.SemaphoreType.DMA,
        ],
    )
    def _sc_gather(pa_hbm, pb_hbm, dst_hbm, src_hbm, ga_hbm, gb_hbm,
                   idx_v, rows_v, sem):
        wid = lax.axis_index("s") * NC + lax.axis_index("c")
        base0 = wid * EPW
        for c in range(NCHUNK):
            base = base0 + c * CHUNK
            pltpu.sync_copy(dst_hbm.at[pl.ds(base, CHUNK)], idx_v)
            pltpu.async_copy(pa_hbm.at[idx_v], rows_v, sem).wait()
            pltpu.sync_copy(rows_v, ga_hbm.at[pl.ds(base, CHUNK)])
            pltpu.sync_copy(src_hbm.at[pl.ds(base, CHUNK)], idx_v)
            pltpu.async_copy(pb_hbm.at[idx_v], rows_v, sem).wait()
            pltpu.sync_copy(rows_v, gb_hbm.at[pl.ds(base, CHUNK)])

    return _sc_gather


NPT = -(-((N + NS - 1) // NS) // 8) * 8
NPAD = NPT * NS
CHUNK_S = 200
NCHUNK_S = EPW // CHUNK_S


@functools.cache
def _sc_scatter_add_kernel():
    mesh = plsc.VectorSubcoreMesh(core_axis_name="c", subcore_axis_name="s",
                                  num_cores=NC, num_subcores=NS)

    @functools.partial(
        pl.kernel,
        out_type=jax.ShapeDtypeStruct((NC, NPAD, D), jnp.float32),
        mesh=mesh,
        scratch_types=[
            pltpu.VMEM((CHUNK_S,), jnp.int32),
            pltpu.VMEM((CHUNK_S, D), jnp.float32),
            pltpu.VMEM_SHARED((NPAD, D), jnp.float32),
        ],
    )
    def _sc_scatter_add(edge_hbm, dst_hbm, zeros_hbm, out_hbm,
                        idx_v, rows_v, agg_sh):
        _sc_scatter_add_body(edge_hbm, dst_hbm, zeros_hbm, out_hbm,
                             idx_v, rows_v, agg_sh)

    return _sc_scatter_add


def _sc_scatter_add_body(edge_hbm, dst_hbm, zeros_hbm, out_hbm,
                         idx_v, rows_v, agg_sh):
    cid = lax.axis_index("c")
    sid = lax.axis_index("s")
    pltpu.sync_copy(zeros_hbm.at[pl.ds(sid * NPT, NPT)],
                    agg_sh.at[pl.ds(sid * NPT, NPT)])
    plsc.subcore_barrier()
    wid = cid * NS + sid
    base0 = wid * EPW
    for c in range(NCHUNK_S):
        base = base0 + c * CHUNK_S
        pltpu.sync_copy(dst_hbm.at[pl.ds(base, CHUNK_S)], idx_v)
        pltpu.sync_copy(edge_hbm.at[pl.ds(base, CHUNK_S)], rows_v)
        pltpu.sync_copy(rows_v, agg_sh.at[idx_v], add=True)
    plsc.subcore_barrier()
    pltpu.sync_copy(agg_sh.at[pl.ds(sid * NPT, NPT)],
                    out_hbm.at[cid, pl.ds(sid * NPT, NPT)])


def kernel(node_x, edge_index, edge_attr,
           mW0, mb0, mW1, mb1, mW2, mb2, mg, mB,
           uW0, ub0, uW1, ub1, uW2, ub2, ug, uB):
    src = edge_index[0]
    dst = edge_index[1]

    pa, pb = _node_proj(node_x, mW0[:D], mW0[D:2 * D])
    ga, gb = _sc_gather_kernel()(pa, pb, dst, src)
    edge_new = _edge_mlp(
        ga, gb, edge_attr, mW0[2 * D:], mW1, mW2,
        mb0.reshape(1, D), mb1.reshape(1, D), mb2.reshape(1, D),
        mg.reshape(1, D), mB.reshape(1, D))
    zeros = jnp.zeros((NPAD, D), jnp.float32)
    partials = _sc_scatter_add_kernel()(edge_new, dst, zeros)
    out = _node_update(
        node_x, partials[0, :N], partials[1, :N],
        uW0[:D], uW0[D:], uW1, uW2,
        ub0.reshape(1, D), ub1.reshape(1, D), ub2.reshape(1, D),
        ug.reshape(1, D), uB.reshape(1, D))
    return out


# speedup vs baseline: 1.5549x; 1.5549x over previous
import functools

import jax
import jax.numpy as jnp
from jax import lax
from jax.experimental import pallas as pl
from jax.experimental.pallas import tpu as pltpu
from jax.experimental.pallas import tpu_sc as plsc

N = 10000
E = 320000
D = 128

NC = 2
NS = 16
NW = NC * NS
EPW = E // NW
CHUNK = 400
NCHUNK = EPW // CHUNK


def _bdot(a, b):
    return jnp.dot(a.astype(jnp.bfloat16), b.astype(jnp.bfloat16),
                   preferred_element_type=jnp.float32)


def _node_proj_body(x_ref, wa_ref, wb_ref, pa_ref, pb_ref):
    x = x_ref[...]
    pa_ref[...] = _bdot(x, wa_ref[...])
    pb_ref[...] = _bdot(x, wb_ref[...])


def _node_proj(node_x, wa, wb):
    bn = 1000
    grid = (N // bn,)
    return pl.pallas_call(
        _node_proj_body,
        grid=grid,
        in_specs=[
            pl.BlockSpec((bn, D), lambda i: (i, 0)),
            pl.BlockSpec((D, D), lambda i: (0, 0)),
            pl.BlockSpec((D, D), lambda i: (0, 0)),
        ],
        out_specs=[
            pl.BlockSpec((bn, D), lambda i: (i, 0)),
            pl.BlockSpec((bn, D), lambda i: (i, 0)),
        ],
        out_shape=[
            jax.ShapeDtypeStruct((N, D), jnp.float32),
            jax.ShapeDtypeStruct((N, D), jnp.float32),
        ],
    )(node_x, wa, wb)


def _ln_affine(h, g, b):
    mu = jnp.mean(h, axis=-1, keepdims=True)
    hc = h - mu
    var = jnp.mean(hc * hc, axis=-1, keepdims=True)
    return hc * lax.rsqrt(var + 1e-5) * g + b


def _edge_mlp_body(ga_ref, gb_ref, e_ref, wc_ref, w1_ref, w2_ref,
                   b0_ref, b1_ref, b2_ref, g_ref, beta_ref, out_ref):
    e = e_ref[...]
    h0 = ga_ref[...] + gb_ref[...] + _bdot(e, wc_ref[...]) + b0_ref[...]
    h0 = jnp.maximum(h0, 0.0)
    h1 = jnp.maximum(_bdot(h0, w1_ref[...]) + b1_ref[...], 0.0)
    h2 = _bdot(h1, w2_ref[...]) + b2_ref[...]
    out_ref[...] = e + _ln_affine(h2, g_ref[...], beta_ref[...])


def _edge_mlp(ga, gb, edge_attr, wc, w1, w2, b0, b1, b2, g, beta):
    be = 1600
    grid = (E // be,)
    row = lambda i: (i, 0)
    fix = lambda i: (0, 0)
    return pl.pallas_call(
        _edge_mlp_body,
        grid=grid,
        in_specs=[
            pl.BlockSpec((be, D), row),
            pl.BlockSpec((be, D), row),
            pl.BlockSpec((be, D), row),
            pl.BlockSpec((D, D), fix),
            pl.BlockSpec((D, D), fix),
            pl.BlockSpec((D, D), fix),
            pl.BlockSpec((1, D), fix),
            pl.BlockSpec((1, D), fix),
            pl.BlockSpec((1, D), fix),
            pl.BlockSpec((1, D), fix),
            pl.BlockSpec((1, D), fix),
        ],
        out_specs=pl.BlockSpec((be, D), row),
        out_shape=jax.ShapeDtypeStruct((E, D), jnp.float32),
    )(ga, gb, edge_attr, wc, w1, w2, b0, b1, b2, g, beta)


def _node_update_body(x_ref, p0_ref, p1_ref, wa_ref, wb_ref, w1_ref, w2_ref,
                      b0_ref, b1_ref, b2_ref, g_ref, beta_ref, out_ref):
    x = x_ref[...]
    agg = p0_ref[...] + p1_ref[...]
    h0 = _bdot(x, wa_ref[...]) + _bdot(agg, wb_ref[...]) + b0_ref[...]
    h0 = jnp.maximum(h0, 0.0)
    h1 = jnp.maximum(_bdot(h0, w1_ref[...]) + b1_ref[...], 0.0)
    h2 = _bdot(h1, w2_ref[...]) + b2_ref[...]
    out_ref[...] = x + _ln_affine(h2, g_ref[...], beta_ref[...])


def _node_update(node_x, p0, p1, wa, wb, w1, w2, b0, b1, b2, g, beta):
    bn = 1000
    grid = (N // bn,)
    row = lambda i: (i, 0)
    fix = lambda i: (0, 0)
    return pl.pallas_call(
        _node_update_body,
        grid=grid,
        in_specs=[
            pl.BlockSpec((bn, D), row),
            pl.BlockSpec((bn, D), row),
            pl.BlockSpec((bn, D), row),
            pl.BlockSpec((D, D), fix),
            pl.BlockSpec((D, D), fix),
            pl.BlockSpec((D, D), fix),
            pl.BlockSpec((D, D), fix),
            pl.BlockSpec((1, D), fix),
            pl.BlockSpec((1, D), fix),
            pl.BlockSpec((1, D), fix),
            pl.BlockSpec((1, D), fix),
            pl.BlockSpec((1, D), fix),
        ],
        out_specs=pl.BlockSpec((bn, D), row),
        out_shape=jax.ShapeDtypeStruct((N, D), jnp.float32),
    )(node_x, p0, p1, wa, wb, w1, w2, b0, b1, b2, g, beta)


@functools.cache
def _sc_gather_kernel():
    mesh = plsc.VectorSubcoreMesh(core_axis_name="c", subcore_axis_name="s",
                                  num_cores=NC, num_subcores=NS)

    @functools.partial(
        pl.kernel,
        out_type=[
            jax.ShapeDtypeStruct((E, D), jnp.float32),
            jax.ShapeDtypeStruct((E, D), jnp.float32),
        ],
        mesh=mesh,
        scratch_types=[
            pltpu.VMEM((CHUNK,), jnp.int32),
            pltpu.VMEM((CHUNK,), jnp.int32),
            pltpu.VMEM((CHUNK, D), jnp.float32),
            pltpu.VMEM((CHUNK, D), jnp.float32),
            pltpu.SemaphoreType.DMA,
            pltpu.SemaphoreType.DMA,
            pltpu.SemaphoreType.DMA,
            pltpu.SemaphoreType.DMA,
        ],
    )
    def _sc_gather(pa_hbm, pb_hbm, dst_hbm, src_hbm, ga_hbm, gb_hbm,
                   idx0, idx1, rows0, rows1, gs0, gs1, ws0, ws1):
        wid = lax.axis_index("s") * NC + lax.axis_index("c")
        base0 = wid * EPW
        idx_v = (idx0, idx1)
        rows_v = (rows0, rows1)
        gsem = (gs0, gs1)
        wsem = (ws0, ws1)
        njobs = 2 * NCHUNK
        pend_g = {}
        pend_w = {}

        def job_refs(j):
            base = base0 + (j // 2) * CHUNK
            if j % 2 == 0:
                return dst_hbm, pa_hbm, ga_hbm, base
            return src_hbm, pb_hbm, gb_hbm, base

        for j in range(njobs):
            p = j & 1
            if j >= 2:
                pend_w.pop(j - 2).wait()
            idx_hbm, tbl_hbm, out_hbm, base = job_refs(j)
            pltpu.sync_copy(idx_hbm.at[pl.ds(base, CHUNK)], idx_v[p])
            pend_g[j] = pltpu.async_copy(tbl_hbm.at[idx_v[p]], rows_v[p],
                                         gsem[p])
            if j >= 1:
                q = (j - 1) & 1
                _, _, out_prev, base_prev = job_refs(j - 1)
                pend_g.pop(j - 1).wait()
                pend_w[j - 1] = pltpu.async_copy(
                    rows_v[q], out_prev.at[pl.ds(base_prev, CHUNK)], wsem[q])
        j = njobs - 1
        q = j & 1
        _, _, out_last, base_last = job_refs(j)
        pend_g.pop(j).wait()
        pend_w[j] = pltpu.async_copy(
            rows_v[q], out_last.at[pl.ds(base_last, CHUNK)], wsem[q])
        pend_w.pop(j - 1).wait()
        pend_w.pop(j).wait()

    return _sc_gather


NPT = -(-((N + NS - 1) // NS) // 8) * 8
NPAD = NPT * NS
CHUNK_S = 80
NCHUNK_S = EPW // CHUNK_S


@functools.cache
def _sc_scatter_add_kernel():
    mesh = plsc.VectorSubcoreMesh(core_axis_name="c", subcore_axis_name="s",
                                  num_cores=NC, num_subcores=NS)

    @functools.partial(
        pl.kernel,
        out_type=jax.ShapeDtypeStruct((NC, NPAD, D), jnp.float32),
        mesh=mesh,
        scratch_types=[
            pltpu.VMEM((NCHUNK_S, CHUNK_S), jnp.int32),
            pltpu.VMEM((CHUNK_S, D), jnp.float32),
            pltpu.VMEM((CHUNK_S, D), jnp.float32),
            pltpu.VMEM_SHARED((NPAD, D), jnp.float32),
            pltpu.SemaphoreType.DMA,
            pltpu.SemaphoreType.DMA,
        ],
    )
    def _sc_scatter_add(edge_hbm, dst3_hbm, zeros_hbm, out_hbm,
                        idx_all, rows0, rows1, agg_sh, ls0, ls1):
        cid = lax.axis_index("c")
        sid = lax.axis_index("s")
        wid = cid * NS + sid
        base0 = wid * EPW
        pltpu.sync_copy(dst3_hbm.at[wid], idx_all)
        pltpu.sync_copy(zeros_hbm.at[pl.ds(sid * NPT, NPT)],
                        agg_sh.at[pl.ds(sid * NPT, NPT)])
        plsc.subcore_barrier()
        rows_v = (rows0, rows1)
        lsem = (ls0, ls1)
        pend = {0: pltpu.async_copy(
            edge_hbm.at[pl.ds(base0, CHUNK_S)], rows_v[0], lsem[0])}
        for c in range(NCHUNK_S):
            p = c & 1
            if c + 1 < NCHUNK_S:
                pend[c + 1] = pltpu.async_copy(
                    edge_hbm.at[pl.ds(base0 + (c + 1) * CHUNK_S, CHUNK_S)],
                    rows_v[(c + 1) & 1], lsem[(c + 1) & 1])
            pend.pop(c).wait()
            pltpu.sync_copy(rows_v[p], agg_sh.at[idx_all.at[c]], add=True)
        plsc.subcore_barrier()
        pltpu.sync_copy(agg_sh.at[pl.ds(sid * NPT, NPT)],
                        out_hbm.at[cid, pl.ds(sid * NPT, NPT)])

    return _sc_scatter_add


def kernel(node_x, edge_index, edge_attr,
           mW0, mb0, mW1, mb1, mW2, mb2, mg, mB,
           uW0, ub0, uW1, ub1, uW2, ub2, ug, uB):
    src = edge_index[0]
    dst = edge_index[1]

    pa, pb = _node_proj(node_x, mW0[:D], mW0[D:2 * D])
    ga, gb = _sc_gather_kernel()(pa, pb, dst, src)
    edge_new = _edge_mlp(
        ga, gb, edge_attr, mW0[2 * D:], mW1, mW2,
        mb0.reshape(1, D), mb1.reshape(1, D), mb2.reshape(1, D),
        mg.reshape(1, D), mB.reshape(1, D))
    zeros = jnp.zeros((NPAD, D), jnp.float32)
    dst3 = dst.reshape(NW, NCHUNK_S, CHUNK_S)
    partials = _sc_scatter_add_kernel()(edge_new, dst3, zeros)
    out = _node_update(
        node_x, partials[0, :N], partials[1, :N],
        uW0[:D], uW0[D:], uW1, uW2,
        ub0.reshape(1, D), ub1.reshape(1, D), ub2.reshape(1, D),
        ug.reshape(1, D), uB.reshape(1, D))
    return out


# speedup vs baseline: 1.6505x; 1.0615x over previous
import functools

import jax
import jax.numpy as jnp
from jax import lax
from jax.experimental import pallas as pl
from jax.experimental.pallas import tpu as pltpu
from jax.experimental.pallas import tpu_sc as plsc

N = 10000
E = 320000
D = 128
NH = 2
E2 = E // NH

NC = 2
NS = 16
NW = NC * NS
EPW = E2 // NW
CHUNK = 200
NCHUNK = EPW // CHUNK


def _bdot(a, b):
    return jnp.dot(a.astype(jnp.bfloat16), b.astype(jnp.bfloat16),
                   preferred_element_type=jnp.float32)


def _node_proj_body(x_ref, wa_ref, wb_ref, pa_ref, pb_ref):
    x = x_ref[...]
    pa_ref[...] = _bdot(x, wa_ref[...])
    pb_ref[...] = _bdot(x, wb_ref[...])


def _node_proj(node_x, wa, wb):
    bn = 1000
    grid = (N // bn,)
    return pl.pallas_call(
        _node_proj_body,
        grid=grid,
        in_specs=[
            pl.BlockSpec((bn, D), lambda i: (i, 0)),
            pl.BlockSpec((D, D), lambda i: (0, 0)),
            pl.BlockSpec((D, D), lambda i: (0, 0)),
        ],
        out_specs=[
            pl.BlockSpec((bn, D), lambda i: (i, 0)),
            pl.BlockSpec((bn, D), lambda i: (i, 0)),
        ],
        out_shape=[
            jax.ShapeDtypeStruct((N, D), jnp.float32),
            jax.ShapeDtypeStruct((N, D), jnp.float32),
        ],
    )(node_x, wa, wb)


def _ln_affine(h, g, b):
    mu = jnp.mean(h, axis=-1, keepdims=True)
    hc = h - mu
    var = jnp.mean(hc * hc, axis=-1, keepdims=True)
    return hc * lax.rsqrt(var + 1e-5) * g + b


def _edge_mlp_body(ga_ref, gb_ref, e_ref, wc_ref, w1_ref, w2_ref,
                   b0_ref, b1_ref, b2_ref, g_ref, beta_ref, out_ref):
    e = e_ref[...]
    h0 = ga_ref[...] + gb_ref[...] + _bdot(e, wc_ref[...]) + b0_ref[...]
    h0 = jnp.maximum(h0, 0.0)
    h1 = jnp.maximum(_bdot(h0, w1_ref[...]) + b1_ref[...], 0.0)
    h2 = _bdot(h1, w2_ref[...]) + b2_ref[...]
    out_ref[...] = e + _ln_affine(h2, g_ref[...], beta_ref[...])


BE = 1600


def _edge_mlp(ga, gb, edge_attr, half, wc, w1, w2, b0, b1, b2, g, beta):
    grid = (E2 // BE,)
    off = half * (E2 // BE)
    row = lambda i: (i, 0)
    erow = lambda i: (i + off, 0)
    fix = lambda i: (0, 0)
    return pl.pallas_call(
        _edge_mlp_body,
        grid=grid,
        in_specs=[
            pl.BlockSpec((BE, D), row),
            pl.BlockSpec((BE, D), row),
            pl.BlockSpec((BE, D), erow),
            pl.BlockSpec((D, D), fix),
            pl.BlockSpec((D, D), fix),
            pl.BlockSpec((D, D), fix),
            pl.BlockSpec((1, D), fix),
            pl.BlockSpec((1, D), fix),
            pl.BlockSpec((1, D), fix),
            pl.BlockSpec((1, D), fix),
            pl.BlockSpec((1, D), fix),
        ],
        out_specs=pl.BlockSpec((BE, D), row),
        out_shape=jax.ShapeDtypeStruct((E2, D), jnp.float32),
    )(ga, gb, edge_attr, wc, w1, w2, b0, b1, b2, g, beta)


def _node_update_body(x_ref, p0_ref, p1_ref, p2_ref, p3_ref,
                      wa_ref, wb_ref, w1_ref, w2_ref,
                      b0_ref, b1_ref, b2_ref, g_ref, beta_ref, out_ref):
    x = x_ref[...]
    agg = (p0_ref[...] + p1_ref[...]) + (p2_ref[...] + p3_ref[...])
    h0 = _bdot(x, wa_ref[...]) + _bdot(agg, wb_ref[...]) + b0_ref[...]
    h0 = jnp.maximum(h0, 0.0)
    h1 = jnp.maximum(_bdot(h0, w1_ref[...]) + b1_ref[...], 0.0)
    h2 = _bdot(h1, w2_ref[...]) + b2_ref[...]
    out_ref[...] = x + _ln_affine(h2, g_ref[...], beta_ref[...])


def _node_update(node_x, ps, wa, wb, w1, w2, b0, b1, b2, g, beta):
    bn = 1000
    grid = (N // bn,)
    row = lambda i: (i, 0)
    fix = lambda i: (0, 0)
    return pl.pallas_call(
        _node_update_body,
        grid=grid,
        in_specs=[pl.BlockSpec((bn, D), row)] * 5 + [
            pl.BlockSpec((D, D), fix),
            pl.BlockSpec((D, D), fix),
            pl.BlockSpec((D, D), fix),
            pl.BlockSpec((D, D), fix),
            pl.BlockSpec((1, D), fix),
            pl.BlockSpec((1, D), fix),
            pl.BlockSpec((1, D), fix),
            pl.BlockSpec((1, D), fix),
            pl.BlockSpec((1, D), fix),
        ],
        out_specs=pl.BlockSpec((bn, D), row),
        out_shape=jax.ShapeDtypeStruct((N, D), jnp.float32),
    )(node_x, *ps, wa, wb, w1, w2, b0, b1, b2, g, beta)


@functools.cache
def _sc_gather_kernel(half):
    mesh = plsc.VectorSubcoreMesh(core_axis_name="c", subcore_axis_name="s",
                                  num_cores=NC, num_subcores=NS)

    @functools.partial(
        pl.kernel,
        out_type=[
            jax.ShapeDtypeStruct((E2, D), jnp.float32),
            jax.ShapeDtypeStruct((E2, D), jnp.float32),
        ],
        mesh=mesh,
        scratch_types=[
            pltpu.VMEM((CHUNK,), jnp.int32),
            pltpu.VMEM((CHUNK,), jnp.int32),
            pltpu.VMEM((CHUNK, D), jnp.float32),
            pltpu.VMEM((CHUNK, D), jnp.float32),
            pltpu.SemaphoreType.DMA,
            pltpu.SemaphoreType.DMA,
            pltpu.SemaphoreType.DMA,
            pltpu.SemaphoreType.DMA,
        ],
    )
    def _sc_gather(pa_hbm, pb_hbm, dst_hbm, src_hbm, ga_hbm, gb_hbm,
                   idx0, idx1, rows0, rows1, gs0, gs1, ws0, ws1):
        wid = lax.axis_index("s") * NC + lax.axis_index("c")
        base0 = wid * EPW
        idx_v = (idx0, idx1)
        rows_v = (rows0, rows1)
        gsem = (gs0, gs1)
        wsem = (ws0, ws1)
        njobs = 2 * NCHUNK
        pend_g = {}
        pend_w = {}

        def job_refs(j):
            base = base0 + (j // 2) * CHUNK
            if j % 2 == 0:
                return dst_hbm, pa_hbm, ga_hbm, base
            return src_hbm, pb_hbm, gb_hbm, base

        for j in range(njobs):
            p = j & 1
            if j >= 2:
                pend_w.pop(j - 2).wait()
            idx_hbm, tbl_hbm, out_hbm, base = job_refs(j)
            pltpu.sync_copy(
                idx_hbm.at[pl.ds(half * E2 + base, CHUNK)], idx_v[p])
            pend_g[j] = pltpu.async_copy(tbl_hbm.at[idx_v[p]], rows_v[p],
                                         gsem[p])
            if j >= 1:
                q = (j - 1) & 1
                _, _, out_prev, base_prev = job_refs(j - 1)
                pend_g.pop(j - 1).wait()
                pend_w[j - 1] = pltpu.async_copy(
                    rows_v[q], out_prev.at[pl.ds(base_prev, CHUNK)], wsem[q])
        j = njobs - 1
        q = j & 1
        _, _, out_last, base_last = job_refs(j)
        pend_g.pop(j).wait()
        pend_w[j] = pltpu.async_copy(
            rows_v[q], out_last.at[pl.ds(base_last, CHUNK)], wsem[q])
        pend_w.pop(j - 1).wait()
        pend_w.pop(j).wait()

    return _sc_gather


NPT = -(-((N + NS - 1) // NS) // 8) * 8
NPAD = NPT * NS
CHUNK_S = 40
NCHUNK_S = EPW // CHUNK_S


@functools.cache
def _sc_scatter_add_kernel(half):
    mesh = plsc.VectorSubcoreMesh(core_axis_name="c", subcore_axis_name="s",
                                  num_cores=NC, num_subcores=NS)

    @functools.partial(
        pl.kernel,
        out_type=jax.ShapeDtypeStruct((NC, NPAD, D), jnp.float32),
        mesh=mesh,
        scratch_types=[
            pltpu.VMEM((NCHUNK_S, CHUNK_S), jnp.int32),
            pltpu.VMEM((CHUNK_S, D), jnp.float32),
            pltpu.VMEM((CHUNK_S, D), jnp.float32),
            pltpu.VMEM_SHARED((NPAD, D), jnp.float32),
            pltpu.SemaphoreType.DMA,
            pltpu.SemaphoreType.DMA,
        ],
    )
    def _sc_scatter_add(edge_hbm, dst4_hbm, zeros_hbm, out_hbm,
                        idx_all, rows0, rows1, agg_sh, ls0, ls1):
        cid = lax.axis_index("c")
        sid = lax.axis_index("s")
        wid = cid * NS + sid
        base0 = wid * EPW
        pltpu.sync_copy(dst4_hbm.at[half * NW + wid], idx_all)
        pltpu.sync_copy(zeros_hbm.at[pl.ds(sid * NPT, NPT)],
                        agg_sh.at[pl.ds(sid * NPT, NPT)])
        plsc.subcore_barrier()
        rows_v = (rows0, rows1)
        lsem = (ls0, ls1)
        pend = {0: pltpu.async_copy(
            edge_hbm.at[pl.ds(base0, CHUNK_S)], rows_v[0], lsem[0])}
        for c in range(NCHUNK_S):
            p = c & 1
            if c + 1 < NCHUNK_S:
                pend[c + 1] = pltpu.async_copy(
                    edge_hbm.at[pl.ds(base0 + (c + 1) * CHUNK_S, CHUNK_S)],
                    rows_v[(c + 1) & 1], lsem[(c + 1) & 1])
            pend.pop(c).wait()
            pltpu.sync_copy(rows_v[p], agg_sh.at[idx_all.at[c]], add=True)
        plsc.subcore_barrier()
        pltpu.sync_copy(agg_sh.at[pl.ds(sid * NPT, NPT)],
                        out_hbm.at[cid, pl.ds(sid * NPT, NPT)])

    return _sc_scatter_add


def kernel(node_x, edge_index, edge_attr,
           mW0, mb0, mW1, mb1, mW2, mb2, mg, mB,
           uW0, ub0, uW1, ub1, uW2, ub2, ug, uB):
    src = edge_index[0]
    dst = edge_index[1]

    pa, pb = _node_proj(node_x, mW0[:D], mW0[D:2 * D])
    zeros = jnp.zeros((NPAD, D), jnp.float32)
    dst4 = dst.reshape(NH * NW, NCHUNK_S, CHUNK_S)
    eb = (mb0.reshape(1, D), mb1.reshape(1, D), mb2.reshape(1, D),
          mg.reshape(1, D), mB.reshape(1, D))

    partials = []
    for h in range(NH):
        ga, gb = _sc_gather_kernel(h)(pa, pb, dst, src)
        edge_new = _edge_mlp(ga, gb, edge_attr, h, mW0[2 * D:], mW1, mW2, *eb)
        partials.append(_sc_scatter_add_kernel(h)(edge_new, dst4, zeros))

    ps = [p[c, :N] for p in partials for c in range(NC)]
    out = _node_update(
        node_x, ps, uW0[:D], uW0[D:], uW1, uW2,
        ub0.reshape(1, D), ub1.reshape(1, D), ub2.reshape(1, D),
        ug.reshape(1, D), uB.reshape(1, D))
    return out
